# baseline (speedup 1.0000x reference)
import functools

import jax
import jax.numpy as jnp
from jax import lax
from jax.experimental import pallas as pl
from jax.experimental.pallas import tpu as pltpu
from jax.experimental.pallas import tpu_sc as plsc

D_MODEL = 128
N_TOKENS = 4096 * 50
NUM_CORES = 2
NUM_SUBCORES = 16
NW = NUM_CORES * NUM_SUBCORES
TOK_PER_W = N_TOKENS // NW
ROWS = 80
N_CHUNKS = TOK_PER_W // ROWS
NBUF = 5


@functools.partial(
    pl.kernel,
    mesh=plsc.VectorSubcoreMesh(core_axis_name="c", subcore_axis_name="s"),
    out_type=jax.ShapeDtypeStruct((N_TOKENS, D_MODEL), jnp.float32),
    scratch_types=(
        [pltpu.VMEM((N_CHUNKS, ROWS), jnp.int32)]
        + [pltpu.VMEM((ROWS, D_MODEL), jnp.float32) for _ in range(NBUF)]
        + [pltpu.VMEM_SHARED((NUM_SUBCORES, NBUF, ROWS, D_MODEL), jnp.float32)]
        + [pltpu.SemaphoreType.DMA for _ in range(3 * NBUF)]
    ),
)
def _embed_gather(table_hbm, idx_hbm, out_hbm, idx_v, *rest):
    bufs = rest[:NBUF]
    spm = rest[NBUF]
    gsem = rest[NBUF + 1:2 * NBUF + 1]
    xsem = rest[2 * NBUF + 1:3 * NBUF + 1]
    dsem = rest[3 * NBUF + 1:]
    cid = lax.axis_index("c")
    sid = lax.axis_index("s")
    wid = sid * NUM_CORES + cid
    base = wid * TOK_PER_W

    def gather(c, b):
        pltpu.make_async_copy(table_hbm.at[idx_v.at[c]], bufs[b], gsem[b]).start()

    def wait_gather(b):
        pltpu.make_async_copy(table_hbm.at[idx_v.at[0]], bufs[b], gsem[b]).wait()

    def xcopy(b):
        pltpu.make_async_copy(bufs[b], spm.at[sid, b], xsem[b]).start()

    def wait_xcopy(b):
        pltpu.make_async_copy(bufs[b], spm.at[sid, b], xsem[b]).wait()

    def drain(c, b):
        pltpu.make_async_copy(
            spm.at[sid, b], out_hbm.at[pl.ds(base + c * ROWS, ROWS)], dsem[b]
        ).start()

    def wait_drain(b):
        pltpu.make_async_copy(
            spm.at[sid, b], out_hbm.at[pl.ds(base, ROWS)], dsem[b]
        ).wait()

    pltpu.sync_copy(idx_hbm.at[wid], idx_v)

    for b in range(NBUF):
        gather(b, b)

    def step(c, j, r_dyn):
        wait_gather(j)
        if r_dyn or not isinstance(c, int) or c >= NBUF:
            wait_drain(j)
        xcopy(j)
        if (not isinstance(c, int)) or c >= 1:
            j1 = (j - 1) % NBUF
            wait_xcopy(j1)
            drain(c - 1, j1)
            c_next = c - 1 + NBUF

            @pl.when(c_next < N_CHUNKS)
            def _():
                gather(c_next, j1)

    for j in range(NBUF):
        step(j, j, r_dyn=False)

    def round_body(r, carry):
        for j in range(NBUF):
            step(r * NBUF + j, j, r_dyn=True)
        return carry

    lax.fori_loop(1, N_CHUNKS // NBUF, round_body, 0)

    wait_xcopy(NBUF - 1)
    drain(N_CHUNKS - 1, NBUF - 1)
    for b in range(NBUF):
        wait_drain(b)


def kernel(x, weight):
    idx = x.reshape(NW, N_CHUNKS, ROWS).astype(jnp.int32)
    out = _embed_gather(weight, idx)
    return out.reshape(x.shape[0], x.shape[1], D_MODEL)


# speedup vs baseline: 1.0000x; 1.0000x over previous
import functools

import jax
import jax.numpy as jnp
from jax import lax
from jax.experimental import pallas as pl
from jax.experimental.pallas import tpu as pltpu
from jax.experimental.pallas import tpu_sc as plsc

D_MODEL = 128
N_TOKENS = 4096 * 50
NUM_CORES = 2
NUM_SUBCORES = 16
NW = NUM_CORES * NUM_SUBCORES
TOK_PER_W = N_TOKENS // NW
ROWS = 80
N_CHUNKS = TOK_PER_W // ROWS
NBUF = 5


@functools.partial(
    pl.kernel,
    mesh=plsc.VectorSubcoreMesh(core_axis_name="c", subcore_axis_name="s"),
    out_type=jax.ShapeDtypeStruct((N_TOKENS, D_MODEL), jnp.float32),
    compiler_params=pltpu.CompilerParams(use_tc_tiling_on_sc=True),
    scratch_types=(
        [pltpu.VMEM((TOK_PER_W,), jnp.int32)]
        + [pltpu.VMEM((ROWS, D_MODEL), jnp.float32) for _ in range(NBUF)]
        + [pltpu.VMEM_SHARED((NUM_SUBCORES, NBUF, ROWS, D_MODEL), jnp.float32)]
        + [pltpu.SemaphoreType.DMA for _ in range(3 * NBUF)]
    ),
)
def _embed_gather(table_hbm, idx_hbm, out_hbm, idx_v, *rest):
    bufs = rest[:NBUF]
    spm = rest[NBUF]
    gsem = rest[NBUF + 1:2 * NBUF + 1]
    xsem = rest[2 * NBUF + 1:3 * NBUF + 1]
    dsem = rest[3 * NBUF + 1:]
    cid = lax.axis_index("c")
    sid = lax.axis_index("s")
    wid = sid * NUM_CORES + cid
    base = wid * TOK_PER_W

    def gather(c, b):
        pltpu.make_async_copy(
            table_hbm.at[idx_v.at[pl.ds(c * ROWS, ROWS)]], bufs[b], gsem[b]
        ).start()

    def wait_gather(b):
        pltpu.make_async_copy(
            table_hbm.at[idx_v.at[pl.ds(0, ROWS)]], bufs[b], gsem[b]
        ).wait()

    def xcopy(b):
        pltpu.make_async_copy(bufs[b], spm.at[sid, b], xsem[b]).start()

    def wait_xcopy(b):
        pltpu.make_async_copy(bufs[b], spm.at[sid, b], xsem[b]).wait()

    def drain(c, b):
        pltpu.make_async_copy(
            spm.at[sid, b], out_hbm.at[pl.ds(base + c * ROWS, ROWS)], dsem[b]
        ).start()

    def wait_drain(b):
        pltpu.make_async_copy(
            spm.at[sid, b], out_hbm.at[pl.ds(base, ROWS)], dsem[b]
        ).wait()

    pltpu.sync_copy(
        idx_hbm.at[pl.ds(pl.multiple_of(wid * TOK_PER_W, 8), TOK_PER_W)], idx_v
    )

    for b in range(NBUF):
        gather(b, b)

    def step(c, j, r_dyn):
        wait_gather(j)
        if r_dyn or not isinstance(c, int) or c >= NBUF:
            wait_drain(j)
        xcopy(j)
        if (not isinstance(c, int)) or c >= 1:
            j1 = (j - 1) % NBUF
            wait_xcopy(j1)
            drain(c - 1, j1)
            c_next = c - 1 + NBUF

            @pl.when(c_next < N_CHUNKS)
            def _():
                gather(c_next, j1)

    for j in range(NBUF):
        step(j, j, r_dyn=False)

    def round_body(r, carry):
        for j in range(NBUF):
            step(r * NBUF + j, j, r_dyn=True)
        return carry

    lax.fori_loop(1, N_CHUNKS // NBUF, round_body, 0)

    wait_xcopy(NBUF - 1)
    drain(N_CHUNKS - 1, NBUF - 1)
    for b in range(NBUF):
        wait_drain(b)


def kernel(x, weight):
    idx = x.reshape(-1).astype(jnp.int32)
    out = _embed_gather(weight, idx)
    return out.reshape(x.shape[0], x.shape[1], D_MODEL)


# speedup vs baseline: 1.7673x; 1.7672x over previous
import functools

import jax
import jax.numpy as jnp
from jax import lax
from jax.experimental import pallas as pl
from jax.experimental.pallas import tpu as pltpu
from jax.experimental.pallas import tpu_sc as plsc

D_MODEL = 128
N_BATCH = 4096
SEQ = 50
NUM_CORES = 2
NUM_SUBCORES = 16
NW = NUM_CORES * NUM_SUBCORES
B_PER_W = N_BATCH // NW
NBUF = 4
HALF = 4
N_ROUNDS = B_PER_W // (2 * HALF)


@functools.partial(
    pl.kernel,
    mesh=plsc.VectorSubcoreMesh(core_axis_name="c", subcore_axis_name="s"),
    out_type=jax.ShapeDtypeStruct((N_BATCH, SEQ, D_MODEL), jnp.float32),
    compiler_params=pltpu.CompilerParams(use_tc_tiling_on_sc=True),
    scratch_types=(
        [pltpu.VMEM((B_PER_W, SEQ), jnp.int32)]
        + [pltpu.VMEM((SEQ, D_MODEL), jnp.float32) for _ in range(NBUF)]
        + [pltpu.VMEM_SHARED((NUM_SUBCORES, 2, HALF, SEQ, D_MODEL), jnp.float32)]
        + [pltpu.SemaphoreType.DMA for _ in range(2 * NBUF + 2)]
    ),
)
def _embed_gather(table_hbm, idx_hbm, out_hbm, idx_v, *rest):
    bufs = rest[:NBUF]
    spm = rest[NBUF]
    gsem = rest[NBUF + 1:2 * NBUF + 1]
    xsem = rest[2 * NBUF + 1:3 * NBUF + 1]
    dsem = rest[3 * NBUF + 1:]
    cid = lax.axis_index("c")
    sid = lax.axis_index("s")
    wid = sid * NUM_CORES + cid
    base_b = wid * B_PER_W

    def gather(c, b):
        pltpu.make_async_copy(table_hbm.at[idx_v.at[c]], bufs[b], gsem[b]).start()

    def wait_gather(b):
        pltpu.make_async_copy(table_hbm.at[idx_v.at[0]], bufs[b], gsem[b]).wait()

    def xcopy(b, h):
        pltpu.make_async_copy(bufs[b], spm.at[sid, h, b], xsem[b]).start()

    def wait_xcopy(b, h):
        pltpu.make_async_copy(bufs[b], spm.at[sid, h, b], xsem[b]).wait()

    def drain(d, h):
        pltpu.make_async_copy(
            spm.at[sid, h], out_hbm.at[pl.ds(base_b + d * HALF, HALF)], dsem[h]
        ).start()

    def wait_drain(h):
        pltpu.make_async_copy(
            spm.at[sid, h], out_hbm.at[pl.ds(base_b, HALF)], dsem[h]
        ).wait()

    pltpu.sync_copy(idx_hbm.at[pl.ds(pl.multiple_of(base_b, 8), B_PER_W)], idx_v)

    for b in range(NBUF):
        gather(b, b)

    def step(c, jj, first_round):
        j = jj % NBUF
        h = (jj // NBUF) % 2
        wait_gather(j)
        if not first_round and j == 0:
            wait_drain(h)
        xcopy(j, h)
        if first_round and jj == 0:
            return
        j1 = (j - 1) % NBUF
        h1 = h if j != 0 else 1 - h
        wait_xcopy(j1, h1)
        if j == 0:
            drain(c // HALF - 1, 1 - h)
        c_next = c - 1 + NBUF

        @pl.when(c_next < B_PER_W)
        def _():
            gather(c_next, j1)

    for jj in range(2 * NBUF):
        step(jj, jj, True)

    def round_body(r, carry):
        for jj in range(2 * NBUF):
            step(r * (2 * NBUF) + jj, jj, False)
        return carry

    lax.fori_loop(1, N_ROUNDS, round_body, 0)

    wait_xcopy(NBUF - 1, 1)
    drain(B_PER_W // HALF - 1, 1)
    for h in range(2):
        wait_drain(h)


def kernel(x, weight):
    return _embed_gather(weight, x.astype(jnp.int32))
